# Initial kernel scaffold; baseline (speedup 1.0000x reference)
import jax
import jax.numpy as jnp
from jax.experimental import pallas as pl


def kernel(features, edge_index, W, b):
    raise NotImplementedError("write your pallas kernel here")


# speedup vs baseline: 30.8766x; 30.8766x over previous
import functools
import jax
import jax.numpy as jnp
from jax import lax
from jax.experimental import pallas as pl
from jax.experimental.pallas import tpu as pltpu
from jax.experimental.pallas import tpu_sc as plsc

N = 10000
E = 320000
D = 128
NP = 10240
EP = 327680
NC = 2
NS = 16
NW = NC * NS
CHUNK = 128
NCHUNK = EP // (NW * CHUNK)
ROWS_PER_TILE = NP // NS

_sc_mesh = functools.partial(
    plsc.VectorSubcoreMesh, core_axis_name="c", subcore_axis_name="s")


def _deg_body(cols_hbm, zeros_hbm, deg_hbm, idx_v, ones_v, deg_sh):
  cid = lax.axis_index("c")
  sid = lax.axis_index("s")
  wid = cid * NS + sid
  for g in range(CHUNK // 16):
    ones_v[pl.ds(g * 16, 16)] = jnp.ones((16,), jnp.float32)
  pltpu.sync_copy(zeros_hbm.at[pl.ds(sid * ROWS_PER_TILE, ROWS_PER_TILE)],
                  deg_sh.at[pl.ds(sid * ROWS_PER_TILE, ROWS_PER_TILE)])
  pltpu.sync_copy(cols_hbm.at[wid], idx_v)
  plsc.subcore_barrier()

  def body(j, _):
    pltpu.sync_copy(ones_v, deg_sh.at[idx_v.at[j]], add=True)
    return 0

  lax.fori_loop(0, NCHUNK, body, 0)
  plsc.subcore_barrier()

  @pl.when(sid == 0)
  def _():
    pltpu.sync_copy(deg_sh, deg_hbm.at[cid])


@jax.jit
def _deg_kernel(cols3, zeros1d):
  return pl.kernel(
      _deg_body,
      out_type=jax.ShapeDtypeStruct((NC, NP), jnp.float32),
      mesh=_sc_mesh(),
      scratch_types=[
          pltpu.VMEM((NCHUNK, CHUNK), jnp.int32),
          pltpu.VMEM((CHUNK,), jnp.float32),
          pltpu.VMEM_SHARED((NP,), jnp.float32),
      ],
  )(cols3, zeros1d)


def _matmul_body(f_ref, w_ref, degp_ref, y_ref):
  deg = degp_ref[0, :] + degp_ref[1, :] + 1.0
  dinv = lax.rsqrt(deg)
  x = jnp.dot(f_ref[...], w_ref[...], preferred_element_type=jnp.float32)
  y_ref[...] = x * dinv[:, None]


@jax.jit
def _matmul_kernel(features_pad, W, deg_p):
  blk = 1024
  return pl.pallas_call(
      _matmul_body,
      grid=(NP // blk,),
      in_specs=[
          pl.BlockSpec((blk, D), lambda i: (i, 0)),
          pl.BlockSpec((D, D), lambda i: (0, 0)),
          pl.BlockSpec((NC, blk), lambda i: (0, i)),
      ],
      out_specs=pl.BlockSpec((blk, D), lambda i: (i, 0)),
      out_shape=jax.ShapeDtypeStruct((NP, D), jnp.float32),
  )(features_pad, W, deg_p)


def _agg_body(y_hbm, rows_hbm, cols_hbm, zeros_hbm, out_hbm,
              idxr_v, idxc_v, rows_v, acc_sh, sem):
  cid = lax.axis_index("c")
  sid = lax.axis_index("s")
  wid = cid * NS + sid
  r0 = sid * ROWS_PER_TILE
  pltpu.sync_copy(zeros_hbm.at[pl.ds(r0, ROWS_PER_TILE)],
                  acc_sh.at[pl.ds(r0, ROWS_PER_TILE)])
  pltpu.sync_copy(rows_hbm.at[wid], idxr_v)
  pltpu.sync_copy(cols_hbm.at[wid], idxc_v)
  plsc.subcore_barrier()

  def body(j, _):
    pltpu.async_copy(y_hbm.at[idxr_v.at[j]], rows_v, sem).wait()
    pltpu.sync_copy(rows_v, acc_sh.at[idxc_v.at[j]], add=True)
    return 0

  lax.fori_loop(0, NCHUNK, body, 0)
  plsc.subcore_barrier()
  pltpu.sync_copy(acc_sh.at[pl.ds(r0, ROWS_PER_TILE)],
                  out_hbm.at[cid, pl.ds(r0, ROWS_PER_TILE)])


@jax.jit
def _agg_kernel(y, rows3, cols3, zeros2d):
  return pl.kernel(
      _agg_body,
      out_type=jax.ShapeDtypeStruct((NC, NP, D), jnp.float32),
      mesh=_sc_mesh(),
      scratch_types=[
          pltpu.VMEM((NCHUNK, CHUNK), jnp.int32),
          pltpu.VMEM((NCHUNK, CHUNK), jnp.int32),
          pltpu.VMEM((CHUNK, D), jnp.float32),
          pltpu.VMEM_SHARED((NP, D), jnp.float32),
          pltpu.SemaphoreType.DMA,
      ],
  )(y, rows3, cols3, zeros2d)


def _combine_body(p_ref, y_ref, degp_ref, b_ref, o_ref):
  deg = degp_ref[0, :] + degp_ref[1, :] + 1.0
  dinv = lax.rsqrt(deg)
  s = p_ref[0] + p_ref[1] + y_ref[...]
  o_ref[...] = s * dinv[:, None] + b_ref[...]


@jax.jit
def _combine_kernel(partials, y, deg_p, b2d):
  blk = 1024
  return pl.pallas_call(
      _combine_body,
      grid=(NP // blk,),
      in_specs=[
          pl.BlockSpec((NC, blk, D), lambda i: (0, i, 0)),
          pl.BlockSpec((blk, D), lambda i: (i, 0)),
          pl.BlockSpec((NC, blk), lambda i: (0, i)),
          pl.BlockSpec((1, D), lambda i: (0, 0)),
      ],
      out_specs=pl.BlockSpec((blk, D), lambda i: (i, 0)),
      out_shape=jax.ShapeDtypeStruct((NP, D), jnp.float32),
  )(partials, y, deg_p, b2d)


def kernel(features, edge_index, W, b):
  row = edge_index[0]
  col = edge_index[1]
  npad = EP - E
  pad_idx = N + (jnp.arange(npad, dtype=jnp.int32) % (NP - N))
  rows3 = jnp.concatenate([row, pad_idx]).reshape(NW, NCHUNK, CHUNK)
  cols3 = jnp.concatenate([col, pad_idx]).reshape(NW, NCHUNK, CHUNK)
  features_pad = jnp.pad(features, ((0, NP - N), (0, 0)))
  zeros1d = jnp.zeros((NP,), jnp.float32)
  zeros2d = jnp.zeros((NP, D), jnp.float32)

  deg_p = _deg_kernel(cols3, zeros1d)
  y = _matmul_kernel(features_pad, W, deg_p)
  partials = _agg_kernel(y, rows3, cols3, zeros2d)
  out = _combine_kernel(partials, y, deg_p, b.reshape(1, D))
  return out[:N]


# speedup vs baseline: 44.1236x; 1.4290x over previous
import functools
import jax
import jax.numpy as jnp
from jax import lax
from jax.experimental import pallas as pl
from jax.experimental.pallas import tpu as pltpu
from jax.experimental.pallas import tpu_sc as plsc

N = 10000
E = 320000
D = 128
NP = 10240
EP = 327680
NC = 2
NS = 16
NW = NC * NS
CHUNK = 128
NCHUNK = EP // (NW * CHUNK)
ROWS_PER_TILE = NP // NS

_sc_mesh = functools.partial(
    plsc.VectorSubcoreMesh, core_axis_name="c", subcore_axis_name="s")


def _deg_body(cols_hbm, zeros_hbm, deg_hbm, idx_v, ones_v, deg_sh, sem):
  cid = lax.axis_index("c")
  sid = lax.axis_index("s")
  wid = cid * NS + sid
  for g in range(CHUNK // 16):
    ones_v[pl.ds(g * 16, 16)] = jnp.ones((16,), jnp.float32)
  pltpu.sync_copy(zeros_hbm.at[pl.ds(sid * ROWS_PER_TILE, ROWS_PER_TILE)],
                  deg_sh.at[pl.ds(sid * ROWS_PER_TILE, ROWS_PER_TILE)])
  pltpu.sync_copy(cols_hbm.at[wid], idx_v)
  plsc.subcore_barrier()

  LAG = 4

  def body(j, _):
    pltpu.async_copy(ones_v, deg_sh.at[idx_v.at[j]], sem, add=True)

    @pl.when(j >= LAG)
    def _():
      pltpu.make_async_copy(ones_v, deg_sh.at[idx_v.at[j - LAG]], sem).wait()

    return 0

  lax.fori_loop(0, NCHUNK, body, 0)

  def drain(j, _):
    pltpu.make_async_copy(ones_v, deg_sh.at[idx_v.at[j]], sem).wait()
    return 0

  lax.fori_loop(NCHUNK - LAG, NCHUNK, drain, 0)
  plsc.subcore_barrier()

  @pl.when(sid == 0)
  def _():
    pltpu.sync_copy(deg_sh, deg_hbm.at[cid])


@jax.jit
def _deg_kernel(cols3, zeros1d):
  return pl.kernel(
      _deg_body,
      out_type=jax.ShapeDtypeStruct((NC, NP), jnp.float32),
      mesh=_sc_mesh(),
      scratch_types=[
          pltpu.VMEM((NCHUNK, CHUNK), jnp.int32),
          pltpu.VMEM((CHUNK,), jnp.float32),
          pltpu.VMEM_SHARED((NP,), jnp.float32),
          pltpu.SemaphoreType.DMA,
      ],
  )(cols3, zeros1d)


def _matmul_body(f_ref, w_ref, degp_ref, y_ref):
  deg = degp_ref[0, :] + degp_ref[1, :] + 1.0
  dinv = lax.rsqrt(deg)
  x = jnp.dot(f_ref[...], w_ref[...], preferred_element_type=jnp.float32)
  y_ref[...] = x * dinv[:, None]


@jax.jit
def _matmul_kernel(features_pad, W, deg_p):
  blk = 1024
  return pl.pallas_call(
      _matmul_body,
      grid=(NP // blk,),
      in_specs=[
          pl.BlockSpec((blk, D), lambda i: (i, 0)),
          pl.BlockSpec((D, D), lambda i: (0, 0)),
          pl.BlockSpec((NC, blk), lambda i: (0, i)),
      ],
      out_specs=pl.BlockSpec((blk, D), lambda i: (i, 0)),
      out_shape=jax.ShapeDtypeStruct((NP, D), jnp.float32),
  )(features_pad, W, deg_p)


def _agg_body(y_hbm, rows_hbm, cols_hbm, zeros_hbm, out_hbm,
              ir_v, ic_v, rows0_v, rows1_v, acc_sh,
              irs0, irs1, ics0, ics1, gsem0, gsem1):
  cid = lax.axis_index("c")
  sid = lax.axis_index("s")
  wid = cid * NS + sid
  r0 = sid * ROWS_PER_TILE
  pltpu.sync_copy(zeros_hbm.at[pl.ds(r0, ROWS_PER_TILE)],
                  acc_sh.at[pl.ds(r0, ROWS_PER_TILE)])

  rows = (rows0_v, rows1_v)
  irsems = (irs0, irs1)
  icsems = (ics0, ics1)
  gsems = (gsem0, gsem1)

  def fetch_ir(j, s):
    pltpu.async_copy(rows_hbm.at[wid, j], ir_v.at[s], irsems[s])

  def wait_ir(j, s):
    pltpu.make_async_copy(rows_hbm.at[wid, j], ir_v.at[s], irsems[s]).wait()

  def fetch_ic(j, s):
    pltpu.async_copy(cols_hbm.at[wid, j], ic_v.at[s], icsems[s])

  def wait_ic(j, s):
    pltpu.make_async_copy(cols_hbm.at[wid, j], ic_v.at[s], icsems[s]).wait()

  def start_gather(s):
    pltpu.async_copy(y_hbm.at[ir_v.at[s]], rows[s], gsems[s])

  def wait_gather(s):
    pltpu.make_async_copy(y_hbm.at[ir_v.at[s]], rows[s], gsems[s]).wait()

  def scatter(s):
    pltpu.sync_copy(rows[s], acc_sh.at[ic_v.at[s]], add=True)

  def process(j, s, t):
    wait_ir(j + 1, t)
    start_gather(t)
    wait_gather(s)
    fetch_ir(j + 2, s)
    wait_ic(j, s)
    scatter(s)
    fetch_ic(j + 2, s)

  fetch_ir(0, 0)
  fetch_ic(0, 0)
  fetch_ir(1, 1)
  fetch_ic(1, 1)
  plsc.subcore_barrier()
  wait_ir(0, 0)
  start_gather(0)

  def body2(i, _):
    j0 = 2 * i
    process(j0, 0, 1)
    process(j0 + 1, 1, 0)
    return 0

  lax.fori_loop(0, NCHUNK // 2 - 1, body2, 0)
  wait_ir(NCHUNK - 1, 1)
  start_gather(1)
  wait_gather(0)
  wait_ic(NCHUNK - 2, 0)
  scatter(0)
  wait_gather(1)
  wait_ic(NCHUNK - 1, 1)
  scatter(1)
  plsc.subcore_barrier()
  pltpu.sync_copy(acc_sh.at[pl.ds(r0, ROWS_PER_TILE)],
                  out_hbm.at[cid, pl.ds(r0, ROWS_PER_TILE)])


@jax.jit
def _agg_kernel(y, rows3, cols3, zeros2d):
  return pl.kernel(
      _agg_body,
      out_type=jax.ShapeDtypeStruct((NC, NP, D), jnp.float32),
      mesh=_sc_mesh(),
      scratch_types=[
          pltpu.VMEM((2, CHUNK), jnp.int32),
          pltpu.VMEM((2, CHUNK), jnp.int32),
          pltpu.VMEM((CHUNK, D), jnp.float32),
          pltpu.VMEM((CHUNK, D), jnp.float32),
          pltpu.VMEM_SHARED((NP, D), jnp.float32),
          pltpu.SemaphoreType.DMA,
          pltpu.SemaphoreType.DMA,
          pltpu.SemaphoreType.DMA,
          pltpu.SemaphoreType.DMA,
          pltpu.SemaphoreType.DMA,
          pltpu.SemaphoreType.DMA,
      ],
  )(y, rows3, cols3, zeros2d)


def _combine_body(p_ref, y_ref, degp_ref, b_ref, o_ref):
  deg = degp_ref[0, :] + degp_ref[1, :] + 1.0
  dinv = lax.rsqrt(deg)
  s = p_ref[0] + p_ref[1] + y_ref[...]
  o_ref[...] = s * dinv[:, None] + b_ref[...]


@jax.jit
def _combine_kernel(partials, y, deg_p, b2d):
  blk = 1024
  return pl.pallas_call(
      _combine_body,
      grid=(NP // blk,),
      in_specs=[
          pl.BlockSpec((NC, blk, D), lambda i: (0, i, 0)),
          pl.BlockSpec((blk, D), lambda i: (i, 0)),
          pl.BlockSpec((NC, blk), lambda i: (0, i)),
          pl.BlockSpec((1, D), lambda i: (0, 0)),
      ],
      out_specs=pl.BlockSpec((blk, D), lambda i: (i, 0)),
      out_shape=jax.ShapeDtypeStruct((NP, D), jnp.float32),
  )(partials, y, deg_p, b2d)


def kernel(features, edge_index, W, b):
  row = edge_index[0]
  col = edge_index[1]
  npad = EP - E
  pad_idx = N + (jnp.arange(npad, dtype=jnp.int32) % (NP - N))
  rows3 = jnp.concatenate([row, pad_idx]).reshape(NW, NCHUNK, CHUNK)
  cols3 = jnp.concatenate([col, pad_idx]).reshape(NW, NCHUNK, CHUNK)
  features_pad = jnp.pad(features, ((0, NP - N), (0, 0)))
  zeros1d = jnp.zeros((NP,), jnp.float32)
  zeros2d = jnp.zeros((NP, D), jnp.float32)

  deg_p = _deg_kernel(cols3, zeros1d)
  y = _matmul_kernel(features_pad, W, deg_p)
  partials = _agg_kernel(y, rows3, cols3, zeros2d)
  out = _combine_kernel(partials, y, deg_p, b.reshape(1, D))
  return out[:N]


# speedup vs baseline: 50.0546x; 1.1344x over previous
import functools
import jax
import jax.numpy as jnp
from jax import lax
from jax.experimental import pallas as pl
from jax.experimental.pallas import tpu as pltpu
from jax.experimental.pallas import tpu_sc as plsc

N = 10000
E = 320000
D = 128
NP = 10240
EP = 327680
NC = 2
NS = 16
NW = NC * NS
CHUNK = 128
NCHUNK = EP // (NW * CHUNK)
ACC_ROWS = 10112
ACC_PER_TILE = ACC_ROWS // NS

_sc_mesh = functools.partial(
    plsc.VectorSubcoreMesh, core_axis_name="c", subcore_axis_name="s")


def _deg_body(cols_hbm, zeros_hbm, deg_hbm, idx_v, ones_v, deg_sh, sem):
  cid = lax.axis_index("c")
  sid = lax.axis_index("s")
  wid = cid * NS + sid
  for g in range(CHUNK // 16):
    ones_v[pl.ds(g * 16, 16)] = jnp.ones((16,), jnp.float32)
  pltpu.sync_copy(zeros_hbm.at[pl.ds(sid * (NP // NS), NP // NS)],
                  deg_sh.at[pl.ds(sid * (NP // NS), NP // NS)])
  pltpu.sync_copy(cols_hbm.at[wid], idx_v)
  plsc.subcore_barrier()

  LAG = 4

  def body(j, _):
    pltpu.async_copy(ones_v, deg_sh.at[idx_v.at[j]], sem, add=True)

    @pl.when(j >= LAG)
    def _():
      pltpu.make_async_copy(ones_v, deg_sh.at[idx_v.at[j - LAG]], sem).wait()

    return 0

  lax.fori_loop(0, NCHUNK, body, 0)

  def drain(j, _):
    pltpu.make_async_copy(ones_v, deg_sh.at[idx_v.at[j]], sem).wait()
    return 0

  lax.fori_loop(NCHUNK - LAG, NCHUNK, drain, 0)
  plsc.subcore_barrier()

  @pl.when(sid == 0)
  def _():
    pltpu.sync_copy(deg_sh, deg_hbm.at[cid])


@jax.jit
def _deg_kernel(cols3, zeros1d):
  return pl.kernel(
      _deg_body,
      out_type=jax.ShapeDtypeStruct((NC, NP), jnp.float32),
      mesh=_sc_mesh(),
      scratch_types=[
          pltpu.VMEM((NCHUNK, CHUNK), jnp.int32),
          pltpu.VMEM((CHUNK,), jnp.float32),
          pltpu.VMEM_SHARED((NP,), jnp.float32),
          pltpu.SemaphoreType.DMA,
      ],
  )(cols3, zeros1d)


def _matmul_body(f_ref, w_ref, degp_ref, y_ref):
  deg = degp_ref[0, :] + degp_ref[1, :] + 1.0
  dinv = lax.rsqrt(deg)
  x = jnp.dot(f_ref[...], w_ref[...], preferred_element_type=jnp.float32)
  y_ref[...] = x * dinv[:, None]


@jax.jit
def _matmul_kernel(features_pad, W, deg_p):
  blk = 2048
  return pl.pallas_call(
      _matmul_body,
      grid=(NP // blk,),
      in_specs=[
          pl.BlockSpec((blk, D), lambda i: (i, 0)),
          pl.BlockSpec((D, D), lambda i: (0, 0)),
          pl.BlockSpec((NC, blk), lambda i: (0, i)),
      ],
      out_specs=pl.BlockSpec((blk, D), lambda i: (i, 0)),
      out_shape=jax.ShapeDtypeStruct((NP, D), jnp.float32),
  )(features_pad, W, deg_p)


def _agg_body(y_hbm, rows_hbm, cols_hbm, out_hbm,
              ir_v, ic_v, rows0_v, rows1_v, rows2_v, acc_sh,
              irs0, irs1, irs2, ics0, ics1, gs0, gs1, gs2):
  cid = lax.axis_index("c")
  sid = lax.axis_index("s")
  wid = cid * NS + sid
  rows = (rows0_v, rows1_v, rows2_v)
  irsems = (irs0, irs1, irs2)
  icsems = (ics0, ics1)
  gsems = (gs0, gs1, gs2)

  z16 = jnp.zeros((16,), jnp.float32)

  def zbody(k, _):
    for g in range(D // 16):
      rows0_v[k, pl.ds(g * 16, 16)] = z16
    return 0

  lax.fori_loop(0, CHUNK, zbody, 0)
  a0 = sid * ACC_PER_TILE
  off = 0
  while off < ACC_PER_TILE:
    step = min(CHUNK, ACC_PER_TILE - off)
    pltpu.sync_copy(rows0_v.at[pl.ds(0, step)],
                    acc_sh.at[pl.ds(a0 + off, step)])
    off += step

  def fetch_ir(j, s):
    pltpu.async_copy(rows_hbm.at[wid, j], ir_v.at[s], irsems[s])

  def wait_ir(j, s):
    pltpu.make_async_copy(rows_hbm.at[wid, j], ir_v.at[s], irsems[s]).wait()

  def fetch_ic(j, s):
    pltpu.async_copy(cols_hbm.at[wid, j], ic_v.at[s], icsems[s])

  def wait_ic(j, s):
    pltpu.make_async_copy(cols_hbm.at[wid, j], ic_v.at[s], icsems[s]).wait()

  def start_gather(s):
    pltpu.async_copy(y_hbm.at[ir_v.at[s]], rows[s], gsems[s])

  def wait_gather(s):
    pltpu.make_async_copy(y_hbm.at[ir_v.at[s]], rows[s], gsems[s]).wait()

  def scatter(s, c):
    pltpu.sync_copy(rows[s], acc_sh.at[ic_v.at[c]], add=True)

  def process(j, jd):
    s, s2, s3 = j % 3, (j + 2) % 3, (j + 3) % 3
    c = j % 2
    wait_ir(jd + 2, s2)
    start_gather(s2)
    wait_gather(s)
    if j + 3 < NCHUNK:
      fetch_ir(jd + 3, s3)
    wait_ic(jd, c)
    scatter(s, c)
    if j + 2 < NCHUNK:
      fetch_ic(jd + 2, c)

  fetch_ir(0, 0)
  fetch_ic(0, 0)
  fetch_ir(1, 1)
  fetch_ic(1, 1)
  fetch_ir(2, 2)
  plsc.subcore_barrier()
  wait_ir(0, 0)
  start_gather(0)
  wait_ir(1, 1)
  start_gather(1)

  UNROLL = 6
  STEADY = 72

  def body6(i, _):
    j0 = UNROLL * i
    for o in range(UNROLL):
      process(o, j0 + o)
    return 0

  lax.fori_loop(0, STEADY // UNROLL, body6, 0)
  for j in range(STEADY, NCHUNK - 2):
    process(j, j)
  for j in range(NCHUNK - 2, NCHUNK):
    wait_gather(j % 3)
    wait_ic(j, j % 2)
    scatter(j % 3, j % 2)
  plsc.subcore_barrier()
  pltpu.sync_copy(acc_sh.at[pl.ds(a0, ACC_PER_TILE)],
                  out_hbm.at[cid, pl.ds(a0, ACC_PER_TILE)])


@jax.jit
def _agg_kernel(y, rows3, cols3):
  return pl.kernel(
      _agg_body,
      out_type=jax.ShapeDtypeStruct((NC, ACC_ROWS, D), jnp.float32),
      mesh=_sc_mesh(),
      scratch_types=[
          pltpu.VMEM((3, CHUNK), jnp.int32),
          pltpu.VMEM((2, CHUNK), jnp.int32),
          pltpu.VMEM((CHUNK, D), jnp.float32),
          pltpu.VMEM((CHUNK, D), jnp.float32),
          pltpu.VMEM((CHUNK, D), jnp.float32),
          pltpu.VMEM_SHARED((ACC_ROWS, D), jnp.float32),
      ] + [pltpu.SemaphoreType.DMA] * 8,
  )(y, rows3, cols3)


def _combine_body(p_ref, y_ref, degp_ref, b_ref, o_ref):
  deg = degp_ref[0, :, 0] + degp_ref[1, :, 0] + 1.0
  dinv = lax.rsqrt(deg)
  s = p_ref[0] + p_ref[1] + y_ref[...]
  o_ref[...] = s * dinv[:, None] + b_ref[...]


@jax.jit
def _combine_kernel(partials, y, deg_p, b2d):
  blk = 1000
  return pl.pallas_call(
      _combine_body,
      grid=(N // blk,),
      in_specs=[
          pl.BlockSpec((NC, blk, D), lambda i: (0, i, 0)),
          pl.BlockSpec((blk, D), lambda i: (i, 0)),
          pl.BlockSpec((NC, blk, 1), lambda i: (0, i, 0)),
          pl.BlockSpec((1, D), lambda i: (0, 0)),
      ],
      out_specs=pl.BlockSpec((blk, D), lambda i: (i, 0)),
      out_shape=jax.ShapeDtypeStruct((N, D), jnp.float32),
  )(partials, y, deg_p.reshape(NC, NP, 1), b2d)


def kernel(features, edge_index, W, b):
  row = edge_index[0]
  col = edge_index[1]
  npad = EP - E
  spread = jnp.arange(npad, dtype=jnp.int32) % (NP - N)
  rows3 = jnp.concatenate([row, N + spread]).reshape(NW, NCHUNK, CHUNK)
  cols_agg3 = jnp.concatenate([col, spread]).reshape(NW, NCHUNK, CHUNK)
  cols_deg3 = jnp.concatenate([col, N + spread]).reshape(NW, NCHUNK, CHUNK)
  features_pad = jnp.pad(features, ((0, NP - N), (0, 0)))
  zeros1d = jnp.zeros((NP,), jnp.float32)

  deg_p = _deg_kernel(cols_deg3, zeros1d)
  y = _matmul_kernel(features_pad, W, deg_p)
  partials = _agg_kernel(y, rows3, cols_agg3)
  return _combine_kernel(partials, y, deg_p, b.reshape(1, D))


# speedup vs baseline: 53.1927x; 1.0627x over previous
import functools
import jax
import jax.numpy as jnp
from jax import lax
from jax.experimental import pallas as pl
from jax.experimental.pallas import tpu as pltpu
from jax.experimental.pallas import tpu_sc as plsc

N = 10000
E = 320000
D = 128
NP = 10240
EP = 327680
NC = 2
NS = 16
NW = NC * NS
CHUNK = 128
NCHUNK = EP // (NW * CHUNK)
ACC_ROWS = 10112
ACC_PER_TILE = ACC_ROWS // NS

_sc_mesh = functools.partial(
    plsc.VectorSubcoreMesh, core_axis_name="c", subcore_axis_name="s")


def _deg_body(edges_hbm, deg_hbm, idx_v, ones_v, zeros_v, deg_sh, sem):
  cid = lax.axis_index("c")
  sid = lax.axis_index("s")
  wid = cid * NS + sid
  for g in range(CHUNK // 16):
    ones_v[pl.ds(g * 16, 16)] = jnp.ones((16,), jnp.float32)
  z16 = jnp.zeros((16,), jnp.float32)

  def zbody(k, _):
    zeros_v[pl.ds(k * 16, 16)] = z16
    return 0

  lax.fori_loop(0, (NP // NS) // 16, zbody, 0)
  pltpu.sync_copy(zeros_v, deg_sh.at[pl.ds(sid * (NP // NS), NP // NS)])
  pltpu.sync_copy(edges_hbm.at[pl.ds(wid * NCHUNK, NCHUNK)], idx_v)
  plsc.subcore_barrier()

  LAG = 4

  def body(j, _):
    pltpu.async_copy(ones_v, deg_sh.at[idx_v.at[j, 1]], sem, add=True)

    @pl.when(j >= LAG)
    def _():
      pltpu.make_async_copy(ones_v, deg_sh.at[idx_v.at[j - LAG, 1]],
                            sem).wait()

    return 0

  lax.fori_loop(0, NCHUNK, body, 0)

  def drain(j, _):
    pltpu.make_async_copy(ones_v, deg_sh.at[idx_v.at[j, 1]], sem).wait()
    return 0

  lax.fori_loop(NCHUNK - LAG, NCHUNK, drain, 0)
  plsc.subcore_barrier()

  @pl.when(sid == 0)
  def _():
    pltpu.sync_copy(deg_sh, deg_hbm.at[cid])


@jax.jit
def _deg_kernel(edges):
  return pl.kernel(
      _deg_body,
      out_type=jax.ShapeDtypeStruct((NC, NP), jnp.float32),
      mesh=_sc_mesh(),
      scratch_types=[
          pltpu.VMEM((NCHUNK, 2, CHUNK), jnp.int32),
          pltpu.VMEM((CHUNK,), jnp.float32),
          pltpu.VMEM((NP // NS,), jnp.float32),
          pltpu.VMEM_SHARED((NP,), jnp.float32),
          pltpu.SemaphoreType.DMA,
      ],
  )(edges)


def _matmul_body(f_ref, w_ref, degp_ref, y_ref):
  deg = degp_ref[0, :] + degp_ref[1, :] + 1.0
  dinv = lax.rsqrt(deg)
  x = jnp.dot(f_ref[...], w_ref[...], preferred_element_type=jnp.float32)
  y_ref[...] = x * dinv[:, None]


@jax.jit
def _matmul_kernel(features_pad, W, deg_p):
  blk = 2048
  return pl.pallas_call(
      _matmul_body,
      grid=(NP // blk,),
      in_specs=[
          pl.BlockSpec((blk, D), lambda i: (i, 0)),
          pl.BlockSpec((D, D), lambda i: (0, 0)),
          pl.BlockSpec((NC, blk), lambda i: (0, i)),
      ],
      out_specs=pl.BlockSpec((blk, D), lambda i: (i, 0)),
      out_shape=jax.ShapeDtypeStruct((NP, D), jnp.float32),
  )(features_pad, W, deg_p)


def _agg_body(y_hbm, edges_hbm, out_hbm,
              ir_v, ic_v, rows0_v, rows1_v, rows2_v, acc_sh,
              irs0, irs1, irs2, ics0, ics1, gs0, gs1, gs2):
  cid = lax.axis_index("c")
  sid = lax.axis_index("s")
  wid = cid * NS + sid
  g0 = wid * NCHUNK
  rows = (rows0_v, rows1_v, rows2_v)
  irsems = (irs0, irs1, irs2)
  icsems = (ics0, ics1)
  gsems = (gs0, gs1, gs2)

  z16 = jnp.zeros((16,), jnp.float32)

  def zbody(k, _):
    for g in range(D // 16):
      rows0_v[k, pl.ds(g * 16, 16)] = z16
    return 0

  lax.fori_loop(0, CHUNK, zbody, 0)
  a0 = sid * ACC_PER_TILE
  off = 0
  while off < ACC_PER_TILE:
    step = min(CHUNK, ACC_PER_TILE - off)
    pltpu.sync_copy(rows0_v.at[pl.ds(0, step)],
                    acc_sh.at[pl.ds(a0 + off, step)])
    off += step

  def fetch_ir(j, s):
    pltpu.async_copy(edges_hbm.at[g0 + j, 0], ir_v.at[s], irsems[s])

  def wait_ir(j, s):
    pltpu.make_async_copy(edges_hbm.at[g0 + j, 0], ir_v.at[s],
                          irsems[s]).wait()

  def fetch_ic(j, s):
    pltpu.async_copy(edges_hbm.at[g0 + j, 1], ic_v.at[s], icsems[s])

  def wait_ic(j, s):
    pltpu.make_async_copy(edges_hbm.at[g0 + j, 1], ic_v.at[s],
                          icsems[s]).wait()

  def start_gather(s):
    pltpu.async_copy(y_hbm.at[ir_v.at[s]], rows[s], gsems[s])

  def wait_gather(s):
    pltpu.make_async_copy(y_hbm.at[ir_v.at[s]], rows[s], gsems[s]).wait()

  def scatter(s, c):
    pltpu.sync_copy(rows[s], acc_sh.at[ic_v.at[c]], add=True)

  def process(j, jd):
    s, s2, s3 = j % 3, (j + 2) % 3, (j + 3) % 3
    c = j % 2
    wait_ir(jd + 2, s2)
    start_gather(s2)
    wait_gather(s)
    if j + 3 < NCHUNK:
      fetch_ir(jd + 3, s3)
    wait_ic(jd, c)
    scatter(s, c)
    if j + 2 < NCHUNK:
      fetch_ic(jd + 2, c)

  fetch_ir(0, 0)
  fetch_ic(0, 0)
  fetch_ir(1, 1)
  fetch_ic(1, 1)
  fetch_ir(2, 2)
  plsc.subcore_barrier()
  wait_ir(0, 0)
  start_gather(0)
  wait_ir(1, 1)
  start_gather(1)

  UNROLL = 6
  STEADY = 72

  def body6(i, _):
    j0 = UNROLL * i
    for o in range(UNROLL):
      process(o, j0 + o)
    return 0

  lax.fori_loop(0, STEADY // UNROLL, body6, 0)
  for j in range(STEADY, NCHUNK - 2):
    process(j, j)
  for j in range(NCHUNK - 2, NCHUNK):
    wait_gather(j % 3)
    wait_ic(j, j % 2)
    scatter(j % 3, j % 2)
  plsc.subcore_barrier()
  pltpu.sync_copy(acc_sh.at[pl.ds(a0, ACC_PER_TILE)],
                  out_hbm.at[cid, pl.ds(a0, ACC_PER_TILE)])


@jax.jit
def _agg_kernel(y, edges):
  return pl.kernel(
      _agg_body,
      out_type=jax.ShapeDtypeStruct((NC, ACC_ROWS, D), jnp.float32),
      mesh=_sc_mesh(),
      scratch_types=[
          pltpu.VMEM((3, CHUNK), jnp.int32),
          pltpu.VMEM((2, CHUNK), jnp.int32),
          pltpu.VMEM((CHUNK, D), jnp.float32),
          pltpu.VMEM((CHUNK, D), jnp.float32),
          pltpu.VMEM((CHUNK, D), jnp.float32),
          pltpu.VMEM_SHARED((ACC_ROWS, D), jnp.float32),
      ] + [pltpu.SemaphoreType.DMA] * 8,
  )(y, edges)


def _combine_body(p_ref, y_ref, degp_ref, b_ref, o_ref):
  deg = degp_ref[0, :, 0] + degp_ref[1, :, 0] + 1.0
  dinv = lax.rsqrt(deg)
  s = p_ref[0] + p_ref[1] + y_ref[...]
  o_ref[...] = s * dinv[:, None] + b_ref[...]


@jax.jit
def _combine_kernel(partials, y, deg_p, b2d):
  blk = 1000
  return pl.pallas_call(
      _combine_body,
      grid=(N // blk,),
      in_specs=[
          pl.BlockSpec((NC, blk, D), lambda i: (0, i, 0)),
          pl.BlockSpec((blk, D), lambda i: (i, 0)),
          pl.BlockSpec((NC, blk, 1), lambda i: (0, i, 0)),
          pl.BlockSpec((1, D), lambda i: (0, 0)),
      ],
      out_specs=pl.BlockSpec((blk, D), lambda i: (i, 0)),
      out_shape=jax.ShapeDtypeStruct((N, D), jnp.float32),
  )(partials, y, deg_p.reshape(NC, NP, 1), b2d)


def kernel(features, edge_index, W, b):
  ech = E // CHUNK
  npadc = (EP - E) // CHUNK
  inter = edge_index.reshape(2, ech, CHUNK).transpose(1, 0, 2)
  ar = jnp.arange(npadc * CHUNK, dtype=jnp.int32)
  pad_rows = (N + ar % (NP - N)).reshape(npadc, 1, CHUNK)
  pad_cols = (N + ar % (ACC_ROWS - N)).reshape(npadc, 1, CHUNK)
  edges = jnp.concatenate(
      [inter, jnp.concatenate([pad_rows, pad_cols], axis=1)], axis=0)
  features_pad = jnp.pad(features, ((0, NP - N), (0, 0)))

  deg_p = _deg_kernel(edges)
  y = _matmul_kernel(features_pad, W, deg_p)
  partials = _agg_kernel(y, edges)
  return _combine_kernel(partials, y, deg_p, b.reshape(1, D))
